# baseline (speedup 1.0000x reference)
import functools

import jax
import jax.numpy as jnp
import numpy as np
from jax import lax
from jax.experimental import pallas as pl
from jax.experimental.pallas import tpu as pltpu
from jax.experimental.pallas import tpu_sc as plsc

_N_PTS = 262144
_IN_DIM = 3
_OUT_DIM = 3
_N_LEVELS = 16
_F = 2
_T = 2 ** 19
_MASK = _T - 1
_BASE_RES = 16.0
_AABB_MIN = -0.5
_PLS = float(np.exp2(np.log2(2048.0 * 0.5 / 16.0) / (16.0 - 1.0)))
_RES = [float(np.floor(_BASE_RES * _PLS ** l)) for l in range(_N_LEVELS)]
_P2 = int(np.uint32(2654435761).view(np.int32))
_P3 = int(np.uint32(805459861).view(np.int32))

_NC, _NS = 2, 16
_NW = _NC * _NS
_PW = _N_PTS // _NW
_C = 1024
_NCHUNK = _PW // _C
_G16 = _C // 16
_ENC_DIM = _N_LEVELS * _F


def _enc_body(xyz_h, tab_h, enc_h, xyzv, idx_v, wdup_v,
              rows_v, enc_v, sem):
    cid = lax.axis_index("c")
    sid = lax.axis_index("s")
    wid = sid * _NC + cid
    base0 = wid * _PW
    iota = lax.iota(jnp.int32, 16)
    pair = iota // 2
    feat = iota & 1

    def chunk(ci, carry):
        base = base0 + ci * _C
        pltpu.sync_copy(xyz_h.at[pl.ds(3 * base, 3 * _C)], xyzv)
        for l in range(_N_LEVELS):
            res = _RES[l]
            lofs = l * _T

            def phase_a(g, c2, res=res, lofs=lofs):
                o = g * 16
                e3 = 3 * (o + iota)
                xs = plsc.load_gather(xyzv, [e3])
                ys = plsc.load_gather(xyzv, [e3 + 1])
                zs = plsc.load_gather(xyzv, [e3 + 2])
                sx = (xs - _AABB_MIN) * res
                sy = (ys - _AABB_MIN) * res
                sz = (zs - _AABB_MIN) * res
                px = sx.astype(jnp.int32)
                py = sy.astype(jnp.int32)
                pz = sz.astype(jnp.int32)
                fx = sx - px.astype(jnp.float32)
                fy = sy - py.astype(jnp.float32)
                fz = sz - pz.astype(jnp.float32)
                hx = (px, px + 1)
                hy0 = py * _P2
                hz0 = pz * _P3
                hyz = ((hy0 ^ hz0, (hy0 + _P2) ^ hz0),
                       (hy0 ^ (hz0 + _P3), (hy0 + _P2) ^ (hz0 + _P3)))
                gx = (1.0 - fx, fx)
                gy = (1.0 - fy, fy)
                gz = (1.0 - fz, fz)
                wxy = [[gx[a] * gy[b] for b in range(2)] for a in range(2)]
                for c in range(8):
                    cx, cy, cz = c & 1, (c >> 1) & 1, (c >> 2) & 1
                    e = 2 * (((hx[cx] ^ hyz[cz][cy]) & _MASK) + lofs)
                    wb = (c * _C + o) * 2
                    plsc.store_scatter(idx_v, [wb + 2 * iota], e)
                    plsc.store_scatter(idx_v, [wb + 2 * iota + 1], e + 1)
                    w = wxy[cx][cy] * gz[cz]
                    plsc.store_scatter(wdup_v, [wb + 2 * iota], w)
                    plsc.store_scatter(wdup_v, [wb + 2 * iota + 1], w)
                return c2

            lax.fori_loop(0, _G16, phase_a, 0)
            pltpu.async_copy(tab_h.at[idx_v], rows_v, sem).wait()

            def phase_b(g, c2, l=l):
                o8 = g * 8
                acc = jnp.zeros((16,), jnp.float32)
                for c in range(8):
                    b = (c * _C + o8) * 2
                    acc = acc + wdup_v[pl.ds(b, 16)] * rows_v[pl.ds(b, 16)]
                plsc.store_scatter(enc_v, [o8 + pair, 2 * l + feat], acc)
                return c2

            lax.fori_loop(0, _C // 8, phase_b, 0)
        pltpu.sync_copy(enc_v, enc_h.at[pl.ds(base, _C)])
        return carry

    lax.fori_loop(0, _NCHUNK, chunk, 0)


@functools.cache
def _get_enc_call():
    return pl.kernel(
        _enc_body,
        out_type=jax.ShapeDtypeStruct((_N_PTS, _ENC_DIM), jnp.float32),
        mesh=plsc.VectorSubcoreMesh(core_axis_name="c", subcore_axis_name="s"),
        compiler_params=pltpu.CompilerParams(
            needs_layout_passes=False, use_tc_tiling_on_sc=False),
        scratch_types=[
            pltpu.VMEM((3 * _C,), jnp.float32),
            pltpu.VMEM((16 * _C,), jnp.int32),
            pltpu.VMEM((16 * _C,), jnp.float32),
            pltpu.VMEM((16 * _C,), jnp.float32),
            pltpu.VMEM((_C, _ENC_DIM), jnp.float32),
            pltpu.SemaphoreType.DMA,
        ],
    )


_BN = 2048


def _mlp_body(enc_ref, w1_ref, w2_ref, o_ref):
    h = jnp.dot(enc_ref[...], w1_ref[...], preferred_element_type=jnp.float32)
    h = jnp.maximum(h, 0.0)
    o = jnp.dot(h, w2_ref[...], preferred_element_type=jnp.float32)
    o_ref[...] = jax.nn.sigmoid(o)


_mlp_call = pl.pallas_call(
    _mlp_body,
    grid=(_N_PTS // _BN,),
    in_specs=[
        pl.BlockSpec((_BN, _ENC_DIM), lambda i: (i, 0)),
        pl.BlockSpec((_ENC_DIM, 64), lambda i: (0, 0)),
        pl.BlockSpec((64, _OUT_DIM), lambda i: (0, 0)),
    ],
    out_specs=pl.BlockSpec((_BN, _OUT_DIM), lambda i: (i, 0)),
    out_shape=jax.ShapeDtypeStruct((_N_PTS, _OUT_DIM), jnp.float32),
)


def kernel(x, tables, W1, W2):
    xyz = x.reshape(_N_PTS * _IN_DIM)
    tab = tables.reshape(_N_LEVELS * _T * _F)
    enc = _get_enc_call()(xyz, tab)
    return _mlp_call(enc, W1, W2)


# speedup vs baseline: 4.7704x; 4.7704x over previous
import functools

import jax
import jax.numpy as jnp
import numpy as np
from jax import lax
from jax.experimental import pallas as pl
from jax.experimental.pallas import tpu as pltpu
from jax.experimental.pallas import tpu_sc as plsc

_N_PTS = 262144
_IN_DIM = 3
_OUT_DIM = 3
_N_LEVELS = 16
_F = 2
_T = 2 ** 19
_MASK = _T - 1
_BASE_RES = 16.0
_AABB_MIN = -0.5
_PLS = float(np.exp2(np.log2(2048.0 * 0.5 / 16.0) / (16.0 - 1.0)))
_RES = [float(np.floor(_BASE_RES * _PLS ** l)) for l in range(_N_LEVELS)]
_P2 = int(np.uint32(2654435761).view(np.int32))
_P3 = int(np.uint32(805459861).view(np.int32))

_NC, _NS = 2, 16
_NW = _NC * _NS
_PW = _N_PTS // _NW
_C = 512
_NCHUNK = _PW // _C
_G16 = _C // 16
_ENC_DIM = _N_LEVELS * _F


def _enc_body(xyz_h, tab_h, enc_h, xyzv, idx0, idx1, wd0, wd1, rw0, rw1,
              enc_v, sem0, sem1):
    cid = lax.axis_index("c")
    sid = lax.axis_index("s")
    wid = sid * _NC + cid
    base0 = wid * _PW
    iota = lax.iota(jnp.int32, 16)
    pair = iota // 2
    feat = iota & 1
    idxb = (idx0, idx1)
    wdb = (wd0, wd1)
    rwb = (rw0, rw1)
    semb = (sem0, sem1)

    def phase_a(l, idx_v, wdup_v):
        res = _RES[l]
        lofs = l * _T * _F

        def body(g, c2):
            o = g * 16
            e3 = 3 * (o + iota)
            xs = plsc.load_gather(xyzv, [e3])
            ys = plsc.load_gather(xyzv, [e3 + 1])
            zs = plsc.load_gather(xyzv, [e3 + 2])
            sx = (xs - _AABB_MIN) * res
            sy = (ys - _AABB_MIN) * res
            sz = (zs - _AABB_MIN) * res
            px = sx.astype(jnp.int32)
            py = sy.astype(jnp.int32)
            pz = sz.astype(jnp.int32)
            fx = sx - px.astype(jnp.float32)
            fy = sy - py.astype(jnp.float32)
            fz = sz - pz.astype(jnp.float32)
            hx = (px, px + 1)
            hy0 = py * _P2
            hz0 = pz * _P3
            hyz = ((hy0 ^ hz0, (hy0 + _P2) ^ hz0),
                   (hy0 ^ (hz0 + _P3), (hy0 + _P2) ^ (hz0 + _P3)))
            gx = (1.0 - fx, fx)
            gy = (1.0 - fy, fy)
            gz = (1.0 - fz, fz)
            wxy = [[gx[a] * gy[b] for b in range(2)] for a in range(2)]
            for c in range(8):
                cx, cy, cz = c & 1, (c >> 1) & 1, (c >> 2) & 1
                h = (hx[cx] ^ hyz[cz][cy]) & _MASK
                e = (h + (h & -128)) + lofs
                wb = (c * _C + o) * 2
                plsc.store_scatter(idx_v, [wb + 2 * iota], e)
                plsc.store_scatter(idx_v, [wb + 2 * iota + 1], e + 128)
                w = wxy[cx][cy] * gz[cz]
                plsc.store_scatter(wdup_v, [wb + 2 * iota], w)
                plsc.store_scatter(wdup_v, [wb + 2 * iota + 1], w)
            return c2

        lax.fori_loop(0, _G16, body, 0)

    def phase_b(l, wdup_v, rows_v):
        def body(g, c2):
            o8 = g * 8
            acc = jnp.zeros((16,), jnp.float32)
            for c in range(8):
                b = (c * _C + o8) * 2
                acc = acc + wdup_v[pl.ds(b, 16)] * rows_v[pl.ds(b, 16)]
            plsc.store_scatter(enc_v, [o8 + pair, 2 * l + feat], acc)
            return c2

        lax.fori_loop(0, _C // 8, body, 0)

    def chunk(ci, carry):
        base = base0 + ci * _C
        pltpu.sync_copy(xyz_h.at[pl.ds(3 * base, 3 * _C)], xyzv)
        phase_a(0, idxb[0], wdb[0])
        prev = pltpu.async_copy(tab_h.at[idxb[0]], rwb[0], semb[0])
        for l in range(1, _N_LEVELS):
            p = l & 1
            phase_a(l, idxb[p], wdb[p])
            cur = pltpu.async_copy(tab_h.at[idxb[p]], rwb[p], semb[p])
            prev.wait()
            phase_b(l - 1, wdb[1 - p], rwb[1 - p])
            prev = cur
        prev.wait()
        phase_b(_N_LEVELS - 1, wdb[1], rwb[1])
        pltpu.sync_copy(enc_v, enc_h.at[pl.ds(base, _C)])
        return carry

    lax.fori_loop(0, _NCHUNK, chunk, 0)


@functools.cache
def _get_enc_call():
    return pl.kernel(
        _enc_body,
        out_type=jax.ShapeDtypeStruct((_N_PTS, _ENC_DIM), jnp.float32),
        mesh=plsc.VectorSubcoreMesh(core_axis_name="c", subcore_axis_name="s"),
        compiler_params=pltpu.CompilerParams(
            needs_layout_passes=False, use_tc_tiling_on_sc=False),
        scratch_types=[
            pltpu.VMEM((3 * _C,), jnp.float32),
            pltpu.VMEM((16 * _C,), jnp.int32),
            pltpu.VMEM((16 * _C,), jnp.int32),
            pltpu.VMEM((16 * _C,), jnp.float32),
            pltpu.VMEM((16 * _C,), jnp.float32),
            pltpu.VMEM((16 * _C,), jnp.float32),
            pltpu.VMEM((16 * _C,), jnp.float32),
            pltpu.VMEM((_C, _ENC_DIM), jnp.float32),
            pltpu.SemaphoreType.DMA,
            pltpu.SemaphoreType.DMA,
        ],
    )


_BN = 2048


def _mlp_body(enc_ref, w1_ref, w2_ref, o_ref):
    h = jnp.dot(enc_ref[...], w1_ref[...], preferred_element_type=jnp.float32)
    h = jnp.maximum(h, 0.0)
    o = jnp.dot(h, w2_ref[...], preferred_element_type=jnp.float32)
    o_ref[...] = jax.nn.sigmoid(o)


_mlp_call = pl.pallas_call(
    _mlp_body,
    grid=(_N_PTS // _BN,),
    in_specs=[
        pl.BlockSpec((_BN, _ENC_DIM), lambda i: (i, 0)),
        pl.BlockSpec((_ENC_DIM, 64), lambda i: (0, 0)),
        pl.BlockSpec((64, _OUT_DIM), lambda i: (0, 0)),
    ],
    out_specs=pl.BlockSpec((_BN, _OUT_DIM), lambda i: (i, 0)),
    out_shape=jax.ShapeDtypeStruct((_N_PTS, _OUT_DIM), jnp.float32),
)


def kernel(x, tables, W1, W2):
    xyz = x.reshape(_N_PTS * _IN_DIM)
    tab = (tables.reshape(_N_LEVELS, _T // 128, 128, _F)
           .transpose(0, 1, 3, 2).reshape(_N_LEVELS * _T * _F))
    enc = _get_enc_call()(xyz, tab)
    return _mlp_call(enc, W1, W2)


# speedup vs baseline: 7.8797x; 1.6518x over previous
import functools

import jax
import jax.numpy as jnp
import numpy as np
from jax import lax
from jax.experimental import pallas as pl
from jax.experimental.pallas import tpu as pltpu
from jax.experimental.pallas import tpu_sc as plsc

_N_PTS = 262144
_IN_DIM = 3
_OUT_DIM = 3
_N_LEVELS = 16
_F = 2
_T = 2 ** 19
_MASK = _T - 1
_BASE_RES = 16.0
_AABB_MIN = -0.5
_PLS = float(np.exp2(np.log2(2048.0 * 0.5 / 16.0) / (16.0 - 1.0)))
_RES = [float(np.floor(_BASE_RES * _PLS ** l)) for l in range(_N_LEVELS)]
_P2 = int(np.uint32(2654435761).view(np.int32))
_P3 = int(np.uint32(805459861).view(np.int32))

_NC, _NS = 2, 16
_NW = _NC * _NS
_PW = _N_PTS // _NW
_C = 512
_NCHUNK = _PW // _C
_G16 = _C // 16
_ENC_DIM = _N_LEVELS * _F


_SLAB = _N_LEVELS * _T * _F // _NW
_FCH = 16384


def _fmt_body(tabp_h, tabrm_h, buf_in, buf_out):
    cid = lax.axis_index("c")
    sid = lax.axis_index("s")
    wid = sid * _NC + cid
    base0 = wid * _SLAB
    iota = lax.iota(jnp.int32, 16)

    def ch(k, carry):
        off = base0 + k * _FCH
        pltpu.sync_copy(tabp_h.at[pl.ds(off, _FCH)], buf_in)

        def blk(b, c2):
            bb = b * 256

            def half(i, c3, bb=bb):
                f0 = buf_in[pl.ds(bb + i * 16, 16)]
                f1 = buf_in[pl.ds(bb + 128 + i * 16, 16)]
                d = bb + i * 32 + 2 * iota
                plsc.store_scatter(buf_out, [d >> 3, d & 7], f0)
                plsc.store_scatter(buf_out, [(d + 1) >> 3, (d + 1) & 7], f1)
                return c3

            lax.fori_loop(0, 8, half, 0)
            return c2

        lax.fori_loop(0, _FCH // 256, blk, 0)
        pltpu.sync_copy(buf_out, tabrm_h.at[pl.ds(off // 8, _FCH // 8), :])
        return carry

    lax.fori_loop(0, _SLAB // _FCH, ch, 0)


@functools.cache
def _get_fmt_call():
    return pl.kernel(
        _fmt_body,
        out_type=jax.ShapeDtypeStruct((_N_LEVELS * _T * _F // 8, 8),
                                      jnp.float32),
        mesh=plsc.VectorSubcoreMesh(core_axis_name="c", subcore_axis_name="s"),
        compiler_params=pltpu.CompilerParams(
            needs_layout_passes=False, use_tc_tiling_on_sc=False),
        scratch_types=[
            pltpu.VMEM((_FCH,), jnp.float32),
            pltpu.VMEM((_FCH // 8, 8), jnp.float32),
        ],
    )


def _enc_body(xyz_h, tab_h, enc_h, xyzv, idx0, idx1, cd0, cd1, wd0, wd1,
              rw0, rw1, enc_v, sem0, sem1):
    cid = lax.axis_index("c")
    sid = lax.axis_index("s")
    wid = sid * _NC + cid
    base0 = wid * _PW
    iota = lax.iota(jnp.int32, 16)
    pair = iota // 2
    feat = iota & 1
    idxb = (idx0, idx1)
    cdb = (cd0, cd1)
    wdb = (wd0, wd1)
    rwb = (rw0, rw1)
    semb = (sem0, sem1)

    def phase_a(l, idx_v, col_v, wdup_v):
        res = _RES[l]
        lofs = l * _T

        def body(g, c2):
            o = g * 16
            e3 = 3 * (o + iota)
            xs = plsc.load_gather(xyzv, [e3])
            ys = plsc.load_gather(xyzv, [e3 + 1])
            zs = plsc.load_gather(xyzv, [e3 + 2])
            sx = (xs - _AABB_MIN) * res
            sy = (ys - _AABB_MIN) * res
            sz = (zs - _AABB_MIN) * res
            px = sx.astype(jnp.int32)
            py = sy.astype(jnp.int32)
            pz = sz.astype(jnp.int32)
            fx = sx - px.astype(jnp.float32)
            fy = sy - py.astype(jnp.float32)
            fz = sz - pz.astype(jnp.float32)
            hx = (px, px + 1)
            hy0 = py * _P2
            hz0 = pz * _P3
            hyz = ((hy0 ^ hz0, (hy0 + _P2) ^ hz0),
                   (hy0 ^ (hz0 + _P3), (hy0 + _P2) ^ (hz0 + _P3)))
            gx = (1.0 - fx, fx)
            gy = (1.0 - fy, fy)
            gz = (1.0 - fz, fz)
            wxy = [[gx[a] * gy[b] for b in range(2)] for a in range(2)]
            for c in range(8):
                cx, cy, cz = c & 1, (c >> 1) & 1, (c >> 2) & 1
                h = (hx[cx] ^ hyz[cz][cy]) & _MASK
                g2 = lofs + h
                wb = (c * _C + o) * 2
                plsc.store_scatter(idx_v, [c * _C + o + iota], g2 >> 2)
                ldx = (h & 3) * 2
                plsc.store_scatter(col_v, [wb + 2 * iota], ldx)
                plsc.store_scatter(col_v, [wb + 2 * iota + 1], ldx + 1)
                w = wxy[cx][cy] * gz[cz]
                plsc.store_scatter(wdup_v, [wb + 2 * iota], w)
                plsc.store_scatter(wdup_v, [wb + 2 * iota + 1], w)
            return c2

        lax.fori_loop(0, _G16, body, 0)

    def phase_b(l, col_v, wdup_v, rows_v):
        def body(g, c2):
            o8 = g * 8
            acc = jnp.zeros((16,), jnp.float32)
            for c in range(8):
                b = (c * _C + o8) * 2
                cols = col_v[pl.ds(b, 16)]
                r = plsc.load_gather(rows_v, [c * _C + o8 + pair, cols])
                acc = acc + wdup_v[pl.ds(b, 16)] * r
            plsc.store_scatter(enc_v, [o8 + pair, 2 * l + feat], acc)
            return c2

        lax.fori_loop(0, _C // 8, body, 0)

    def chunk(ci, carry):
        base = base0 + ci * _C
        pltpu.sync_copy(xyz_h.at[pl.ds(3 * base, 3 * _C)], xyzv)
        phase_a(0, idxb[0], cdb[0], wdb[0])
        prev = pltpu.async_copy(tab_h.at[idxb[0]], rwb[0], semb[0])
        for l in range(1, _N_LEVELS):
            p = l & 1
            phase_a(l, idxb[p], cdb[p], wdb[p])
            cur = pltpu.async_copy(tab_h.at[idxb[p]], rwb[p], semb[p])
            prev.wait()
            phase_b(l - 1, cdb[1 - p], wdb[1 - p], rwb[1 - p])
            prev = cur
        prev.wait()
        phase_b(_N_LEVELS - 1, cdb[1], wdb[1], rwb[1])
        pltpu.sync_copy(enc_v, enc_h.at[pl.ds(base, _C)])
        return carry

    lax.fori_loop(0, _NCHUNK, chunk, 0)


@functools.cache
def _get_enc_call():
    return pl.kernel(
        _enc_body,
        out_type=jax.ShapeDtypeStruct((_N_PTS, _ENC_DIM), jnp.float32),
        mesh=plsc.VectorSubcoreMesh(core_axis_name="c", subcore_axis_name="s"),
        compiler_params=pltpu.CompilerParams(
            needs_layout_passes=False, use_tc_tiling_on_sc=False),
        scratch_types=[
            pltpu.VMEM((3 * _C,), jnp.float32),
            pltpu.VMEM((8 * _C,), jnp.int32),
            pltpu.VMEM((8 * _C,), jnp.int32),
            pltpu.VMEM((16 * _C,), jnp.int32),
            pltpu.VMEM((16 * _C,), jnp.int32),
            pltpu.VMEM((16 * _C,), jnp.float32),
            pltpu.VMEM((16 * _C,), jnp.float32),
            pltpu.VMEM((8 * _C, 8), jnp.float32),
            pltpu.VMEM((8 * _C, 8), jnp.float32),
            pltpu.VMEM((_C, _ENC_DIM), jnp.float32),
            pltpu.SemaphoreType.DMA,
            pltpu.SemaphoreType.DMA,
        ],
    )


_BN = 2048


def _mlp_body(enc_ref, w1_ref, w2_ref, o_ref):
    h = jnp.dot(enc_ref[...], w1_ref[...], preferred_element_type=jnp.float32)
    h = jnp.maximum(h, 0.0)
    o = jnp.dot(h, w2_ref[...], preferred_element_type=jnp.float32)
    o_ref[...] = jax.nn.sigmoid(o)


_mlp_call = pl.pallas_call(
    _mlp_body,
    grid=(_N_PTS // _BN,),
    in_specs=[
        pl.BlockSpec((_BN, _ENC_DIM), lambda i: (i, 0)),
        pl.BlockSpec((_ENC_DIM, 64), lambda i: (0, 0)),
        pl.BlockSpec((64, _OUT_DIM), lambda i: (0, 0)),
    ],
    out_specs=pl.BlockSpec((_BN, _OUT_DIM), lambda i: (i, 0)),
    out_shape=jax.ShapeDtypeStruct((_N_PTS, _OUT_DIM), jnp.float32),
)


def kernel(x, tables, W1, W2):
    xyz = x.reshape(_N_PTS * _IN_DIM)
    tabp = (tables.reshape(_N_LEVELS, _T // 128, 128, _F)
            .transpose(0, 1, 3, 2).reshape(_N_LEVELS * _T * _F))
    tab8 = _get_fmt_call()(tabp)
    enc = _get_enc_call()(xyz, tab8)
    return _mlp_call(enc, W1, W2)
